# baseline (speedup 1.0000x reference)
import functools

import jax
import jax.numpy as jnp
from jax import lax
from jax.experimental import pallas as pl
from jax.experimental.pallas import tpu as pltpu
from jax.experimental.pallas import tpu_sc as plsc

VOCAB = 100000
N_EMBD = 1024
CTX = 4096
BATCH = 4
SEQ = 4096

NUM_CORES = 2
NUM_SUBCORES = 16
NUM_WORKERS = NUM_CORES * NUM_SUBCORES
LANES = 16

POS_PER_W = SEQ // NUM_WORKERS
CHUNK = 32
PCHUNKS = POS_PER_W // CHUNK
UNITS = PCHUNKS * BATCH


def _embed_sc(ids_flat, token_table, pos_table):
    mesh = plsc.VectorSubcoreMesh(core_axis_name="c", subcore_axis_name="s")

    @functools.partial(
        pl.kernel,
        out_type=jax.ShapeDtypeStruct((BATCH * SEQ, N_EMBD), jnp.float32),
        mesh=mesh,
        scratch_types=[
            pltpu.VMEM((BATCH * POS_PER_W,), jnp.int32),
            pltpu.VMEM((CHUNK, N_EMBD), jnp.float32),
            pltpu.VMEM((CHUNK, N_EMBD), jnp.float32),
            pltpu.VMEM((CHUNK, N_EMBD), jnp.float32),
            pltpu.SemaphoreType.DMA,
            pltpu.SemaphoreType.DMA,
            pltpu.SemaphoreType.DMA,
            pltpu.SemaphoreType.DMA,
        ],
    )
    def k(ids_hbm, tok_hbm, pos_hbm, out_hbm,
          idx_v, pos_v, rows0, rows1, gsem0, gsem1, osem0, osem1):
        wid = lax.axis_index("s") * NUM_CORES + lax.axis_index("c")
        pbase = wid * POS_PER_W

        for b in range(BATCH):
            pltpu.sync_copy(ids_hbm.at[pl.ds(b * SEQ + pbase, POS_PER_W)],
                            idx_v.at[pl.ds(b * POS_PER_W, POS_PER_W)])
        pltpu.sync_copy(pos_hbm.at[pl.ds(pbase, CHUNK)], pos_v)

        rows = [rows0, rows1]
        gsem = [gsem0, gsem1]
        osem = [osem0, osem1]
        pending_g = [None, None]
        pending_o = [None, None]

        def start_gather(i):
            b, p = i % BATCH, i // BATCH
            buf = i % 2
            idx_sl = idx_v.at[pl.ds(b * POS_PER_W + p * CHUNK, CHUNK)]
            pending_g[buf] = pltpu.async_copy(
                tok_hbm.at[idx_sl], rows[buf], gsem[buf])

        start_gather(0)
        for i in range(UNITS):
            b, p = i % BATCH, i // BATCH
            buf = i % 2
            if i + 1 < UNITS:
                nbuf = (i + 1) % 2
                if pending_o[nbuf] is not None:
                    pending_o[nbuf].wait()
                    pending_o[nbuf] = None
                start_gather(i + 1)
            pending_g[buf].wait()

            rv = rows[buf]

            @plsc.parallel_loop(0, CHUNK * N_EMBD // LANES, unroll=16)
            def _(t, rv=rv):
                r = t >> 6
                sl = pl.ds((t & (N_EMBD // LANES - 1)) * LANES, LANES)
                plsc.addupdate(rv.at[r, sl], pos_v[r, sl])

            if b == BATCH - 1 and i + 1 < UNITS:
                pltpu.sync_copy(
                    pos_hbm.at[pl.ds(pbase + (p + 1) * CHUNK, CHUNK)], pos_v)

            pending_o[buf] = pltpu.async_copy(
                rows[buf],
                out_hbm.at[pl.ds(b * SEQ + pbase + p * CHUNK, CHUNK)],
                osem[buf])

        pending_o[0].wait()
        pending_o[1].wait()

    return k(ids_flat, token_table, pos_table)


@jax.jit
def kernel(token_ids, token_table, pos_table):
    ids_flat = token_ids.reshape(BATCH * SEQ)
    out = _embed_sc(ids_flat, token_table, pos_table)
    return out.reshape(BATCH, SEQ, N_EMBD)


# speedup vs baseline: 1.2048x; 1.2048x over previous
import functools

import jax
import jax.numpy as jnp
from jax import lax
from jax.experimental import pallas as pl
from jax.experimental.pallas import tpu as pltpu
from jax.experimental.pallas import tpu_sc as plsc

VOCAB = 100000
N_EMBD = 1024
CTX = 4096
BATCH = 4
SEQ = 4096

NUM_CORES = 2
NUM_SUBCORES = 16
NUM_WORKERS = NUM_CORES * NUM_SUBCORES
LANES = 16

POS_PER_W = SEQ // NUM_WORKERS
CHUNK = 16
NBUF = 4
PCHUNKS = POS_PER_W // CHUNK
UNITS = PCHUNKS * BATCH
ADD_UNROLL = 8


def _embed_sc(ids_flat, token_table, pos_table):
    mesh = plsc.VectorSubcoreMesh(core_axis_name="c", subcore_axis_name="s")

    @functools.partial(
        pl.kernel,
        out_type=jax.ShapeDtypeStruct((BATCH * SEQ, N_EMBD), jnp.float32),
        mesh=mesh,
        scratch_types=(
            [pltpu.VMEM((BATCH * POS_PER_W,), jnp.int32)]
            + [pltpu.VMEM((CHUNK, N_EMBD), jnp.float32) for _ in range(2)]
            + [pltpu.VMEM((CHUNK, N_EMBD), jnp.float32) for _ in range(NBUF)]
            + [pltpu.SemaphoreType.DMA for _ in range(3 + 2 * NBUF)]
        ),
    )
    def k(ids_hbm, tok_hbm, pos_hbm, out_hbm, idx_v, *bufs_and_sems):
        pos_bufs = list(bufs_and_sems[:2])
        rows = list(bufs_and_sems[2:2 + NBUF])
        sems = list(bufs_and_sems[2 + NBUF:])
        isem, psem0, psem1 = sems[:3]
        psem = [psem0, psem1]
        gsem = sems[3:3 + NBUF]
        osem = sems[3 + NBUF:3 + 2 * NBUF]

        wid = lax.axis_index("s") * NUM_CORES + lax.axis_index("c")
        pbase = wid * POS_PER_W

        pending_pos = [None, None]
        pending_pos[0] = pltpu.async_copy(
            pos_hbm.at[pl.ds(pbase, CHUNK)], pos_bufs[0], psem[0])
        idx_copies = [
            pltpu.async_copy(ids_hbm.at[pl.ds(b * SEQ + pbase, POS_PER_W)],
                             idx_v.at[pl.ds(b * POS_PER_W, POS_PER_W)], isem)
            for b in range(BATCH)
        ]
        for c in idx_copies:
            c.wait()

        pending_g = [None] * NBUF
        pending_o = [None] * NBUF

        def start_gather(j):
            b, p = j % BATCH, j // BATCH
            buf = j % NBUF
            idx_sl = idx_v.at[pl.ds(b * POS_PER_W + p * CHUNK, CHUNK)]
            pending_g[buf] = pltpu.async_copy(
                tok_hbm.at[idx_sl], rows[buf], gsem[buf])

        for j in range(NBUF - 1):
            start_gather(j)

        for i in range(UNITS):
            b, p = i % BATCH, i // BATCH
            buf = i % NBUF

            if b == 0:
                pending_pos[p % 2].wait()
                pending_pos[p % 2] = None
                if p + 1 < PCHUNKS:
                    pending_pos[(p + 1) % 2] = pltpu.async_copy(
                        pos_hbm.at[pl.ds(pbase + (p + 1) * CHUNK, CHUNK)],
                        pos_bufs[(p + 1) % 2], psem[(p + 1) % 2])
            pv = pos_bufs[p % 2]

            pending_g[buf].wait()
            pending_g[buf] = None

            rv = rows[buf]

            @plsc.parallel_loop(0, CHUNK * N_EMBD // LANES, unroll=ADD_UNROLL)
            def _(t, rv=rv, pv=pv):
                r = t >> 6
                sl = pl.ds((t & (N_EMBD // LANES - 1)) * LANES, LANES)
                plsc.addupdate(rv.at[r, sl], pv[r, sl])

            pending_o[buf] = pltpu.async_copy(
                rows[buf],
                out_hbm.at[pl.ds(b * SEQ + pbase + p * CHUNK, CHUNK)],
                osem[buf])

            j = i + NBUF - 1
            if j < UNITS:
                jbuf = j % NBUF
                if pending_o[jbuf] is not None:
                    pending_o[jbuf].wait()
                    pending_o[jbuf] = None
                start_gather(j)

        for buf in range(NBUF):
            if pending_o[buf] is not None:
                pending_o[buf].wait()

    return k(ids_flat, token_table, pos_table)


@jax.jit
def kernel(token_ids, token_table, pos_table):
    ids_flat = token_ids.reshape(BATCH * SEQ)
    out = _embed_sc(ids_flat, token_table, pos_table)
    return out.reshape(BATCH, SEQ, N_EMBD)


# speedup vs baseline: 1.2102x; 1.0045x over previous
import functools

import jax
import jax.numpy as jnp
from jax import lax
from jax.experimental import pallas as pl
from jax.experimental.pallas import tpu as pltpu
from jax.experimental.pallas import tpu_sc as plsc

VOCAB = 100000
N_EMBD = 1024
CTX = 4096
BATCH = 4
SEQ = 4096

NUM_CORES = 2
NUM_SUBCORES = 16
NUM_WORKERS = NUM_CORES * NUM_SUBCORES
LANES = 16

POS_PER_W = SEQ // NUM_WORKERS
CHUNK = 16
NBUF = 5
PCHUNKS = POS_PER_W // CHUNK
UNITS = PCHUNKS * BATCH
ADD_UNROLL = 8


def _embed_sc(ids_flat, token_table, pos_table):
    mesh = plsc.VectorSubcoreMesh(core_axis_name="c", subcore_axis_name="s")

    @functools.partial(
        pl.kernel,
        out_type=jax.ShapeDtypeStruct((BATCH * SEQ, N_EMBD), jnp.float32),
        mesh=mesh,
        scratch_types=(
            [pltpu.VMEM((BATCH * POS_PER_W,), jnp.int32)]
            + [pltpu.VMEM((CHUNK, N_EMBD), jnp.float32) for _ in range(2)]
            + [pltpu.VMEM((CHUNK, N_EMBD), jnp.float32) for _ in range(NBUF)]
            + [pltpu.SemaphoreType.DMA for _ in range(3 + 2 * NBUF)]
        ),
    )
    def k(ids_hbm, tok_hbm, pos_hbm, out_hbm, idx_v, *bufs_and_sems):
        pos_bufs = list(bufs_and_sems[:2])
        rows = list(bufs_and_sems[2:2 + NBUF])
        sems = list(bufs_and_sems[2 + NBUF:])
        isem, psem0, psem1 = sems[:3]
        psem = [psem0, psem1]
        gsem = sems[3:3 + NBUF]
        osem = sems[3 + NBUF:3 + 2 * NBUF]

        wid = lax.axis_index("s") * NUM_CORES + lax.axis_index("c")
        pbase = wid * POS_PER_W

        pending_pos = [None, None]
        pending_pos[0] = pltpu.async_copy(
            pos_hbm.at[pl.ds(pbase, CHUNK)], pos_bufs[0], psem[0])
        idx_copies = [
            pltpu.async_copy(ids_hbm.at[pl.ds(b * SEQ + pbase, POS_PER_W)],
                             idx_v.at[pl.ds(b * POS_PER_W, POS_PER_W)], isem)
            for b in range(BATCH)
        ]
        for c in idx_copies:
            c.wait()

        pending_g = [None] * NBUF
        pending_o = [None] * NBUF

        def start_gather(j):
            b, p = j % BATCH, j // BATCH
            buf = j % NBUF
            idx_sl = idx_v.at[pl.ds(b * POS_PER_W + p * CHUNK, CHUNK)]
            pending_g[buf] = pltpu.async_copy(
                tok_hbm.at[idx_sl], rows[buf], gsem[buf])

        for j in range(NBUF - 1):
            start_gather(j)

        for i in range(UNITS):
            b, p = i % BATCH, i // BATCH
            buf = i % NBUF

            if b == 0:
                pending_pos[p % 2].wait()
                pending_pos[p % 2] = None
                if p + 1 < PCHUNKS:
                    pending_pos[(p + 1) % 2] = pltpu.async_copy(
                        pos_hbm.at[pl.ds(pbase + (p + 1) * CHUNK, CHUNK)],
                        pos_bufs[(p + 1) % 2], psem[(p + 1) % 2])
            pv = pos_bufs[p % 2]

            pending_g[buf].wait()
            pending_g[buf] = None

            rv = rows[buf]

            @plsc.parallel_loop(0, CHUNK * N_EMBD // LANES, unroll=ADD_UNROLL)
            def _(t, rv=rv, pv=pv):
                r = t >> 6
                sl = pl.ds((t & (N_EMBD // LANES - 1)) * LANES, LANES)
                plsc.addupdate(rv.at[r, sl], pv[r, sl])

            pending_o[buf] = pltpu.async_copy(
                rows[buf],
                out_hbm.at[pl.ds(b * SEQ + pbase + p * CHUNK, CHUNK)],
                osem[buf])

            j = i + NBUF - 1
            if j < UNITS:
                jbuf = j % NBUF
                if pending_o[jbuf] is not None:
                    pending_o[jbuf].wait()
                    pending_o[jbuf] = None
                start_gather(j)

        for buf in range(NBUF):
            if pending_o[buf] is not None:
                pending_o[buf].wait()

    return k(ids_flat, token_table, pos_table)


@jax.jit
def kernel(token_ids, token_table, pos_table):
    ids_flat = token_ids.reshape(BATCH * SEQ)
    out = _embed_sc(ids_flat, token_table, pos_table)
    return out.reshape(BATCH, SEQ, N_EMBD)
